# Initial kernel scaffold; baseline (speedup 1.0000x reference)
import jax
import jax.numpy as jnp
from jax.experimental import pallas as pl


def kernel(x, batch, W_ih, W_hh, b_ih, b_hh):
    raise NotImplementedError("write your pallas kernel here")


# speedup vs baseline: 13.7573x; 13.7573x over previous
import jax
import jax.numpy as jnp
from jax.experimental import pallas as pl
from jax.experimental.pallas import tpu as pltpu

N = 100000
D = 128
B = 64
STEPS = 3
BN = 5000
NBLK = N // BN
NEG = -1e30


def _body(x_ref, batch_ref, wih_ref, whh_ref, b_ref, out_ref,
          h_ref, c_ref, qs_ref, m_ref, s_ref, r_ref):
    step = pl.program_id(0)
    blk = pl.program_id(1)

    @pl.when(blk == 0)
    def _start_step():
        @pl.when(step == 0)
        def _init():
            qs_ref[...] = jnp.zeros((B, 2 * D), jnp.float32)
            h_ref[...] = jnp.zeros((B, D), jnp.float32)
            c_ref[...] = jnp.zeros((B, D), jnp.float32)

        @pl.when(step > 0)
        def _finalize_prev():
            s = s_ref[...]
            r = r_ref[...]
            denom = jnp.where(s > 0.0, s, 1.0).reshape(B, 1)
            qs_ref[:, D:] = r / denom
            qs_ref[:, :D] = h_ref[...]

        gates = (
            jnp.dot(qs_ref[...], wih_ref[...], preferred_element_type=jnp.float32)
            + jnp.dot(h_ref[...], whh_ref[...], preferred_element_type=jnp.float32)
            + b_ref[...]
        )
        i_g = jax.nn.sigmoid(gates[:, :D])
        f_g = jax.nn.sigmoid(gates[:, D:2 * D])
        g_g = jnp.tanh(gates[:, 2 * D:3 * D])
        o_g = jax.nn.sigmoid(gates[:, 3 * D:])
        c = f_g * c_ref[...] + i_g * g_g
        c_ref[...] = c
        h_ref[...] = o_g * jnp.tanh(c)

        m_ref[...] = jnp.full((1, B), NEG, jnp.float32)
        s_ref[...] = jnp.zeros((1, B), jnp.float32)
        r_ref[...] = jnp.zeros((B, D), jnp.float32)

    x = x_ref[...]
    q = h_ref[...]
    seg = batch_ref[...]
    lane = jax.lax.broadcasted_iota(jnp.int32, (1, B), 1)
    mask = (seg == lane)
    mask_f = mask.astype(jnp.float32)

    scores = jax.lax.dot_general(
        x, q, (((1,), (1,)), ((), ())), preferred_element_type=jnp.float32
    )
    e = jnp.where(mask, scores, NEG)

    m_old = m_ref[...]
    m_new = jnp.maximum(m_old, jnp.max(e, axis=0, keepdims=True))
    scale = jnp.exp(m_old - m_new)
    p = jnp.exp(e - m_new) * mask_f

    s_ref[...] = s_ref[...] * scale + jnp.sum(p, axis=0, keepdims=True)
    pr = jax.lax.dot_general(
        p, x, (((0,), (0,)), ((), ())), preferred_element_type=jnp.float32
    )
    r_ref[...] = r_ref[...] * scale.reshape(B, 1) + pr
    m_ref[...] = m_new

    @pl.when(jnp.logical_and(step == STEPS - 1, blk == NBLK - 1))
    def _emit():
        s = s_ref[...]
        denom = jnp.where(s > 0.0, s, 1.0).reshape(B, 1)
        out_ref[:, :D] = h_ref[...]
        out_ref[:, D:] = r_ref[...] / denom


def kernel(x, batch, W_ih, W_hh, b_ih, b_hh):
    batch2d = batch.astype(jnp.int32).reshape(N, 1)
    bias = (b_ih + b_hh).reshape(1, 4 * D)
    wih_t = W_ih.T
    whh_t = W_hh.T

    grid = (STEPS, NBLK)
    return pl.pallas_call(
        _body,
        grid=grid,
        in_specs=[
            pl.BlockSpec((BN, D), lambda s, k: (k, 0)),
            pl.BlockSpec((BN, 1), lambda s, k: (k, 0)),
            pl.BlockSpec((2 * D, 4 * D), lambda s, k: (0, 0)),
            pl.BlockSpec((D, 4 * D), lambda s, k: (0, 0)),
            pl.BlockSpec((1, 4 * D), lambda s, k: (0, 0)),
        ],
        out_specs=pl.BlockSpec((B, 2 * D), lambda s, k: (0, 0)),
        out_shape=jax.ShapeDtypeStruct((B, 2 * D), jnp.float32),
        scratch_shapes=[
            pltpu.VMEM((B, D), jnp.float32),
            pltpu.VMEM((B, D), jnp.float32),
            pltpu.VMEM((B, 2 * D), jnp.float32),
            pltpu.VMEM((1, B), jnp.float32),
            pltpu.VMEM((1, B), jnp.float32),
            pltpu.VMEM((B, D), jnp.float32),
        ],
    )(x, batch2d, wih_t, whh_t, bias)


# speedup vs baseline: 14.1146x; 1.0260x over previous
import jax
import jax.numpy as jnp
from jax.experimental import pallas as pl
from jax.experimental.pallas import tpu as pltpu

N = 100000
D = 128
B = 64
STEPS = 3
BN = 5000
NBLK = N // BN
NEG = -1e30


def _body(x_ref, cadd_ref, wih_ref, whh_ref, b_ref, out_ref,
          h_ref, c_ref, qs_ref, m_ref, s_ref, r_ref):
    step = pl.program_id(0)
    blk = pl.program_id(1)

    @pl.when(blk == 0)
    def _start_step():
        @pl.when(step == 0)
        def _init():
            qs_ref[...] = jnp.zeros((B, 2 * D), jnp.float32)
            h_ref[...] = jnp.zeros((B, D), jnp.float32)
            c_ref[...] = jnp.zeros((B, D), jnp.float32)

        @pl.when(step > 0)
        def _finalize_prev():
            s = s_ref[...]
            r = r_ref[...]
            denom = jnp.where(s > 0.0, s, 1.0).reshape(B, 1)
            qs_ref[:, D:] = r / denom
            qs_ref[:, :D] = h_ref[...]

        gates = (
            jnp.dot(qs_ref[...], wih_ref[...], preferred_element_type=jnp.float32)
            + jnp.dot(h_ref[...], whh_ref[...], preferred_element_type=jnp.float32)
            + b_ref[...]
        )
        i_g = jax.nn.sigmoid(gates[:, :D])
        f_g = jax.nn.sigmoid(gates[:, D:2 * D])
        g_g = jnp.tanh(gates[:, 2 * D:3 * D])
        o_g = jax.nn.sigmoid(gates[:, 3 * D:])
        c = f_g * c_ref[...] + i_g * g_g
        c_ref[...] = c
        h_ref[...] = o_g * jnp.tanh(c)

        m_ref[...] = jnp.zeros((1, B), jnp.float32)
        s_ref[...] = jnp.zeros((1, B), jnp.float32)
        r_ref[...] = jnp.zeros((B, D), jnp.float32)

    x = x_ref[...]
    q = h_ref[...]

    scores = jax.lax.dot_general(
        x, q, (((1,), (1,)), ((), ())), preferred_element_type=jnp.float32
    )
    e = scores + cadd_ref[...]

    m_old = m_ref[...]
    m_new = jnp.maximum(m_old, jnp.max(e, axis=0, keepdims=True))
    scale = jnp.exp(m_old - m_new)
    p = jnp.exp(e - m_new)

    s_ref[...] = s_ref[...] * scale + jnp.sum(p, axis=0, keepdims=True)
    pr = jax.lax.dot_general(
        p, x, (((0,), (0,)), ((), ())), preferred_element_type=jnp.float32
    )
    r_ref[...] = r_ref[...] * scale.reshape(B, 1) + pr
    m_ref[...] = m_new

    @pl.when(jnp.logical_and(step == STEPS - 1, blk == NBLK - 1))
    def _emit():
        s = s_ref[...]
        denom = jnp.where(s > 0.0, s, 1.0).reshape(B, 1)
        out_ref[:, :D] = h_ref[...]
        out_ref[:, D:] = r_ref[...] / denom


def kernel(x, batch, W_ih, W_hh, b_ih, b_hh):
    onehot = batch.astype(jnp.int32).reshape(N, 1) == jnp.arange(B, dtype=jnp.int32).reshape(1, B)
    cadd = jnp.where(onehot, 0.0, NEG).astype(jnp.float32)
    bias = (b_ih + b_hh).reshape(1, 4 * D)
    wih_t = W_ih.T
    whh_t = W_hh.T

    grid = (STEPS, NBLK)
    return pl.pallas_call(
        _body,
        grid=grid,
        in_specs=[
            pl.BlockSpec((BN, D), lambda s, k: (k, 0)),
            pl.BlockSpec((BN, B), lambda s, k: (k, 0)),
            pl.BlockSpec((2 * D, 4 * D), lambda s, k: (0, 0)),
            pl.BlockSpec((D, 4 * D), lambda s, k: (0, 0)),
            pl.BlockSpec((1, 4 * D), lambda s, k: (0, 0)),
        ],
        out_specs=pl.BlockSpec((B, 2 * D), lambda s, k: (0, 0)),
        out_shape=jax.ShapeDtypeStruct((B, 2 * D), jnp.float32),
        scratch_shapes=[
            pltpu.VMEM((B, D), jnp.float32),
            pltpu.VMEM((B, D), jnp.float32),
            pltpu.VMEM((B, 2 * D), jnp.float32),
            pltpu.VMEM((1, B), jnp.float32),
            pltpu.VMEM((1, B), jnp.float32),
            pltpu.VMEM((B, D), jnp.float32),
        ],
    )(x, cadd, wih_t, whh_t, bias)
